# baseline (speedup 1.0000x reference)
import functools

import jax
import jax.numpy as jnp
from jax import lax
from jax.experimental import pallas as pl
from jax.experimental.pallas import tpu as pltpu
from jax.experimental.pallas import tpu_sc as plsc

_NUM_CORES = 2
_NUM_SUBCORES = 16
_NW = _NUM_CORES * _NUM_SUBCORES


def kernel(tokens, W_pos):
    batch, seq = tokens.shape
    d = W_pos.shape[1]
    rows_per_w = seq // _NW
    chunks = (32, 24, 32, 24, 16)
    n_chunks = len(chunks)
    starts = [sum(chunks[:i]) for i in range(n_chunks)]

    mesh = plsc.VectorSubcoreMesh(core_axis_name="c", subcore_axis_name="s")

    @functools.partial(
        pl.kernel,
        mesh=mesh,
        out_type=jax.ShapeDtypeStruct((batch, seq, d), W_pos.dtype),
        scratch_types=[
            pltpu.VMEM((32, d), W_pos.dtype),
            pltpu.VMEM((24, d), W_pos.dtype),
            pltpu.SemaphoreType.DMA,
            pltpu.SemaphoreType.DMA,
        ],
    )
    def _copy(w_hbm, out_hbm, buf_a, buf_b, sem_in, sem_out):
        wid = lax.axis_index("s") * _NUM_CORES + lax.axis_index("c")
        base = wid * rows_per_w

        def bufref(ci):
            return (buf_a if ci % 2 == 0 else buf_b).at[pl.ds(0, chunks[ci]), :]

        ins = [None] * n_chunks
        outs = [None] * n_chunks
        ins[0] = pltpu.async_copy(
            w_hbm.at[pl.ds(base + starts[0], chunks[0]), :], bufref(0), sem_in)
        for ci in range(n_chunks):
            ins[ci].wait()
            if ci >= 1:
                for h in outs[ci - 1]:
                    h.wait()
            if ci + 1 < n_chunks:
                ins[ci + 1] = pltpu.async_copy(
                    w_hbm.at[pl.ds(base + starts[ci + 1], chunks[ci + 1]), :],
                    bufref(ci + 1), sem_in)
            outs[ci] = [
                pltpu.async_copy(
                    bufref(ci),
                    out_hbm.at[b, pl.ds(base + starts[ci], chunks[ci]), :],
                    sem_out)
                for b in range(batch)
            ]
        for h in outs[n_chunks - 1]:
            h.wait()

    return _copy(W_pos)


# speedup vs baseline: 1.4174x; 1.4174x over previous
import jax
import jax.numpy as jnp
from jax.experimental import pallas as pl


def _body(w_ref, out_ref):
    out_ref[...] = jnp.broadcast_to(w_ref[...][None], out_ref.shape)


def kernel(tokens, W_pos):
    batch, seq = tokens.shape
    d = W_pos.shape[1]
    s_blk = 512
    return pl.pallas_call(
        _body,
        grid=(seq // s_blk,),
        in_specs=[pl.BlockSpec((s_blk, d), lambda i: (i, 0))],
        out_specs=pl.BlockSpec((batch, s_blk, d), lambda i: (0, i, 0)),
        out_shape=jax.ShapeDtypeStruct((batch, seq, d), W_pos.dtype),
    )(W_pos)
